# baseline (speedup 1.0000x reference)
import functools
import jax
import jax.numpy as jnp
from jax import lax
from jax.experimental import pallas as pl
from jax.experimental.pallas import tpu as pltpu
from jax.experimental.pallas import tpu_sc as plsc

N = 10000
E = 320000
DN = 128
DE = 16
HID = 256
DO = 128
HALF = 128
NSC = 2
NTILES = 16
K = 64
GCHUNKS = E // K
NMAIN = 312
NPAIR = NMAIN // 2
NLEFT = GCHUNKS - NMAIN * NTILES
RPT = 624
NTAIL = N - NTILES * RPT


def _sc_body(xah, xbh, th, row, col, zrows, zdeg, out_h, out_deg,
             rcol0, grow0, gcol0, scol0, rcol1, grow1, gcol1, scol1,
             abuf0, bbuf0, tbuf0, abuf1, bbuf1, tbuf1, onesv,
             acc, degacc,
             semI0, semI1, semA0, semB0, semT0, semA1, semB1, semT1,
             semS0, semS1, semD0, semD1):
    c = lax.axis_index("c")
    s = lax.axis_index("s")

    pltpu.sync_copy(zrows, acc.at[pl.ds(s * RPT, RPT)])

    @pl.when(s == NTILES - 1)
    def _():
        pltpu.sync_copy(zrows.at[pl.ds(0, NTAIL)],
                        acc.at[pl.ds(NTILES * RPT, NTAIL)])

    @pl.when(s == 0)
    def _():
        pltpu.sync_copy(zdeg, degacc)

    def initones(i, _):
        onesv[pl.ds(i * 16, 16)] = jnp.full((16,), 1.0, jnp.float32)
        return 0
    lax.fori_loop(0, K // 16, initones, 0)

    plsc.subcore_barrier()

    tbase = c * E
    noff = c * N

    def idx_start(g, grow, rcol, semI):
        e0 = g * K
        pltpu.async_copy(row.at[pl.ds(e0, K)], grow, semI)
        pltpu.async_copy(col.at[pl.ds(e0, K)], rcol, semI)

    def idx_wait(grow, rcol, semI):
        pltpu.make_async_copy(row.at[pl.ds(0, K)], grow, semI).wait()
        pltpu.make_async_copy(col.at[pl.ds(0, K)], rcol, semI).wait()

    def adjust(grow, rcol, gcol):
        def adj(i, _):
            sl = pl.ds(i * 16, 16)
            grow[sl] = grow[sl] + noff
            gcol[sl] = rcol[sl] + noff
            return 0
        lax.fori_loop(0, K // 16, adj, 0)

    def ab_start(grow, gcol, ab, bb, semA, semB):
        pltpu.async_copy(xah.at[grow], ab, semA)
        pltpu.async_copy(xbh.at[gcol], bb, semB)

    def t_start(g, tb, semT):
        e0 = g * K
        pltpu.async_copy(th.at[pl.ds(tbase + e0, K)], tb, semT)

    def gat_wait(grow, gcol, ab, bb, tb, semA, semB, semT):
        pltpu.make_async_copy(xah.at[grow], ab, semA).wait()
        pltpu.make_async_copy(xbh.at[gcol], bb, semB).wait()
        pltpu.make_async_copy(th.at[pl.ds(tbase, K)], tb, semT).wait()

    def comp(ab, bb, tb):
        def body(r, _):
            for f8 in range(HALF // 16):
                sl = pl.ds(f8 * 16, 16)
                tb[r, sl] = jnp.maximum(ab[r, sl] + bb[r, sl] + tb[r, sl],
                                        0.0)
            return 0
        lax.fori_loop(0, K, body, 0)

    def copy_scol(rc, sc):
        def cp(i, _):
            sl = pl.ds(i * 16, 16)
            sc[sl] = rc[sl]
            return 0
        lax.fori_loop(0, K // 16, cp, 0)

    def scat_start(tb, sc, semS, semD):
        pltpu.async_copy(tb, acc.at[sc], semS, add=True)

        @pl.when(c == 0)
        def _():
            pltpu.async_copy(onesv, degacc.at[sc], semD, add=True)

    def scat_wait(tb, sc, semS, semD):
        pltpu.make_async_copy(tb, acc.at[sc], semS).wait()

        @pl.when(c == 0)
        def _():
            pltpu.make_async_copy(onesv, degacc.at[sc], semD).wait()

    p0 = (rcol0, grow0, gcol0, scol0, abuf0, bbuf0, tbuf0,
          semI0, semA0, semB0, semT0, semS0, semD0)
    p1 = (rcol1, grow1, gcol1, scol1, abuf1, bbuf1, tbuf1,
          semI1, semA1, semB1, semT1, semS1, semD1)

    def half(t, mine, other, joff):
        (rc, gr, gc, sc, ab, bb, tb, sI, sA, sB, sT, sS, sD) = mine
        (orc, ogr, ogc, osc, oab, obb, otb,
         osI, osA, osB, osT, osS, osD) = other
        j = 2 * t + joff
        g_next = (j + 1) * NTILES + s
        g_next2 = (j + 2) * NTILES + s

        @pl.when(j + 1 < NMAIN)
        def _():
            idx_wait(ogr, orc, osI)
            adjust(ogr, orc, ogc)
            ab_start(ogr, ogc, oab, obb, osA, osB)

        gat_wait(gr, gc, ab, bb, tb, sA, sB, sT)
        copy_scol(rc, sc)

        @pl.when(j + 2 < NMAIN)
        def _():
            idx_start(g_next2, gr, rc, sI)

        comp(ab, bb, tb)

        @pl.when(j >= 1)
        def _():
            scat_wait(otb, osc, osS, osD)

        @pl.when(j + 1 < NMAIN)
        def _():
            t_start(g_next, otb, osT)

        scat_start(tb, sc, sS, sD)

    g0 = s
    pltpu.sync_copy(row.at[pl.ds(g0 * K, K)], grow0)
    pltpu.sync_copy(col.at[pl.ds(g0 * K, K)], rcol0)
    adjust(grow0, rcol0, gcol0)
    ab_start(grow0, gcol0, abuf0, bbuf0, semA0, semB0)
    t_start(g0, tbuf0, semT0)
    idx_start(1 * NTILES + s, grow1, rcol1, semI1)

    def pair(t, _):
        half(t, p0, p1, 0)
        half(t, p1, p0, 1)
        return 0
    lax.fori_loop(0, NPAIR, pair, 0)

    scat_wait(tbuf1, scol1, semS1, semD1)

    @pl.when(s < NLEFT)
    def _():
        g = NMAIN * NTILES + s
        pltpu.sync_copy(row.at[pl.ds(g * K, K)], grow0)
        pltpu.sync_copy(col.at[pl.ds(g * K, K)], rcol0)
        adjust(grow0, rcol0, gcol0)
        ab_start(grow0, gcol0, abuf0, bbuf0, semA0, semB0)
        t_start(g, tbuf0, semT0)
        gat_wait(grow0, gcol0, abuf0, bbuf0, tbuf0, semA0, semB0, semT0)
        comp(abuf0, bbuf0, tbuf0)
        pltpu.sync_copy(tbuf0, acc.at[rcol0], add=True)

        @pl.when(c == 0)
        def _():
            pltpu.sync_copy(onesv, degacc.at[rcol0], add=True)

    plsc.subcore_barrier()

    pltpu.sync_copy(acc.at[pl.ds(s * RPT, RPT)],
                    out_h.at[pl.ds(c * N + s * RPT, RPT)])

    @pl.when(s == NTILES - 1)
    def _():
        pltpu.sync_copy(acc.at[pl.ds(NTILES * RPT, NTAIL)],
                        out_h.at[pl.ds(c * N + NTILES * RPT, NTAIL)])

    @pl.when((s == 0) & (c == 0))
    def _():
        pltpu.sync_copy(degacc, out_deg)


_sc_agg = functools.partial(
    pl.kernel,
    out_type=(jax.ShapeDtypeStruct((NSC * N, HALF), jnp.float32),
              jax.ShapeDtypeStruct((N,), jnp.float32)),
    mesh=plsc.VectorSubcoreMesh(core_axis_name="c", subcore_axis_name="s"),
    scratch_types=(
        [pltpu.VMEM((K,), jnp.int32)] * 8
        + [pltpu.VMEM((K, HALF), jnp.float32)] * 6
        + [pltpu.VMEM((K,), jnp.float32)]
        + [pltpu.VMEM_SHARED((N, HALF), jnp.float32),
           pltpu.VMEM_SHARED((N,), jnp.float32)]
        + [pltpu.SemaphoreType.DMA] * 12
    ),
)(_sc_body)


def _node_mm_body(x_ref, wa_ref, wb_ref, oa_ref, ob_ref):
    x = x_ref[...]
    oa_ref[...] = jnp.dot(x, wa_ref[...],
                          preferred_element_type=jnp.float32)[None]
    ob_ref[...] = jnp.dot(x, wb_ref[...],
                          preferred_element_type=jnp.float32)[None]


def _node_mm(x, wa, wb):
    nb = 1000
    return pl.pallas_call(
        _node_mm_body,
        grid=(NSC, N // nb),
        in_specs=[
            pl.BlockSpec((nb, DN), lambda h, i: (i, 0)),
            pl.BlockSpec((DN, HALF), lambda h, i: (0, h)),
            pl.BlockSpec((DN, HALF), lambda h, i: (0, h)),
        ],
        out_specs=[
            pl.BlockSpec((1, nb, HALF), lambda h, i: (h, i, 0)),
            pl.BlockSpec((1, nb, HALF), lambda h, i: (h, i, 0)),
        ],
        out_shape=[
            jax.ShapeDtypeStruct((NSC, N, HALF), jnp.float32),
            jax.ShapeDtypeStruct((NSC, N, HALF), jnp.float32),
        ],
    )(x, wa, wb)


def _edge_mm_body(e_ref, we_ref, b_ref, o_ref):
    o_ref[...] = (jnp.dot(e_ref[...], we_ref[...],
                          preferred_element_type=jnp.float32)[None]
                  + b_ref[...])


def _edge_mm(e, we, b1h):
    eb = 4000
    return pl.pallas_call(
        _edge_mm_body,
        grid=(NSC, E // eb),
        in_specs=[
            pl.BlockSpec((eb, DE), lambda h, i: (i, 0)),
            pl.BlockSpec((DE, HALF), lambda h, i: (0, h)),
            pl.BlockSpec((1, 1, HALF), lambda h, i: (h, 0, 0)),
        ],
        out_specs=pl.BlockSpec((1, eb, HALF), lambda h, i: (h, i, 0)),
        out_shape=jax.ShapeDtypeStruct((NSC, E, HALF), jnp.float32),
    )(e, we, b1h)


def _wt_body(w2_ref, u1b_ref, b2_ref, v_ref, c_ref):
    u1b = u1b_ref[...]
    v_ref[...] = jnp.dot(w2_ref[...], u1b, preferred_element_type=jnp.float32)
    c_ref[...] = jnp.dot(b2_ref[...], u1b, preferred_element_type=jnp.float32)


def _wt_mm(w2, u1b, b2):
    return pl.pallas_call(
        _wt_body,
        out_shape=[
            jax.ShapeDtypeStruct((HID, HID), jnp.float32),
            jax.ShapeDtypeStruct((1, HID), jnp.float32),
        ],
    )(w2, u1b, b2)


def _post_body(x_ref, h0_ref, h1_ref, d_ref, u1a_ref, v0_ref, v1_ref,
               cv_ref, ub1_ref, u2_ref, ub2_ref, o_ref):
    p = (jnp.dot(x_ref[...], u1a_ref[...], preferred_element_type=jnp.float32)
         + jnp.dot(h0_ref[...], v0_ref[...], preferred_element_type=jnp.float32)
         + jnp.dot(h1_ref[...], v1_ref[...], preferred_element_type=jnp.float32)
         + d_ref[...] * cv_ref[...]
         + ub1_ref[...])
    u = jnp.maximum(p, 0.0)
    o_ref[...] = jnp.dot(u, u2_ref[...],
                         preferred_element_type=jnp.float32) + ub2_ref[...]


def _post_mm(x, hs, deg, u1a, v0, v1, cv, ub1, u2, ub2):
    nb = 1000
    return pl.pallas_call(
        _post_body,
        grid=(N // nb,),
        in_specs=[
            pl.BlockSpec((nb, DN), lambda i: (i, 0)),
            pl.BlockSpec((nb, HALF), lambda i: (i, 0)),
            pl.BlockSpec((nb, HALF), lambda i: (N // nb + i, 0)),
            pl.BlockSpec((nb, 1), lambda i: (i, 0)),
            pl.BlockSpec((DN, HID), lambda i: (0, 0)),
            pl.BlockSpec((HALF, HID), lambda i: (0, 0)),
            pl.BlockSpec((HALF, HID), lambda i: (0, 0)),
            pl.BlockSpec((1, HID), lambda i: (0, 0)),
            pl.BlockSpec((1, HID), lambda i: (0, 0)),
            pl.BlockSpec((HID, DO), lambda i: (0, 0)),
            pl.BlockSpec((1, DO), lambda i: (0, 0)),
        ],
        out_specs=pl.BlockSpec((nb, DO), lambda i: (i, 0)),
        out_shape=jax.ShapeDtypeStruct((N, DO), jnp.float32),
    )(x, hs, hs, deg, u1a, v0, v1, cv, ub1, u2, ub2)


def kernel(node_features, edge_features, edge_index, W1, b1, W2, b2,
           U1, ub1, U2, ub2):
    row = edge_index[0]
    col = edge_index[1]
    w1a = W1[:DN]
    w1b = W1[DN:2 * DN]
    w1e = W1[2 * DN:]

    xa, xb = _node_mm(node_features, w1a, w1b)
    th = _edge_mm(edge_features, w1e, b1.reshape(NSC, 1, HALF))
    v, cv = _wt_mm(W2, U1[DN:], b2.reshape(1, HID))

    zrows = jnp.zeros((RPT, HALF), jnp.float32)
    zdeg = jnp.zeros((N,), jnp.float32)

    hs, degf = _sc_agg(xa.reshape(NSC * N, HALF), xb.reshape(NSC * N, HALF),
                       th.reshape(NSC * E, HALF), row, col, zrows, zdeg)

    return _post_mm(node_features, hs, degf.reshape(N, 1),
                    U1[:DN], v[:HALF], v[HALF:], cv, ub1.reshape(1, HID),
                    U2, ub2.reshape(1, DO))


# speedup vs baseline: 1.1485x; 1.1485x over previous
import functools
import jax
import jax.numpy as jnp
from jax import lax
from jax.experimental import pallas as pl
from jax.experimental.pallas import tpu as pltpu
from jax.experimental.pallas import tpu_sc as plsc

N = 10000
E = 320000
DN = 128
DE = 16
HID = 256
DO = 128
HALF = 128
NSC = 2
NTILES = 16
K = 64
GCHUNKS = E // K
NMAIN = 312
NPAIR = NMAIN // 2
NLEFT = GCHUNKS - NMAIN * NTILES
RPT = 624
NTAIL = N - NTILES * RPT


def _sc_body(xah, xbh, th, row, col, zrows, zdeg, out_h, out_deg,
             rcol0, grow0, gcol0, rcol1, grow1, gcol1,
             abuf0, bbuf0, tbuf0, abuf1, bbuf1, tbuf1, onesv,
             acc, degacc,
             semI0, semI1, semA0, semB0, semT0, semA1, semB1, semT1):
    c = lax.axis_index("c")
    s = lax.axis_index("s")

    pltpu.sync_copy(zrows, acc.at[pl.ds(s * RPT, RPT)])

    @pl.when(s == NTILES - 1)
    def _():
        pltpu.sync_copy(zrows.at[pl.ds(0, NTAIL)],
                        acc.at[pl.ds(NTILES * RPT, NTAIL)])

    @pl.when(s == 0)
    def _():
        pltpu.sync_copy(zdeg, degacc)

    def initones(i, _):
        onesv[pl.ds(i * 16, 16)] = jnp.full((16,), 1.0, jnp.float32)
        return 0
    lax.fori_loop(0, K // 16, initones, 0)

    plsc.subcore_barrier()

    tbase = c * E
    noff = c * N

    def idx_start(g, grow, rcol, semI):
        e0 = g * K
        pltpu.async_copy(row.at[pl.ds(e0, K)], grow, semI)
        pltpu.async_copy(col.at[pl.ds(e0, K)], rcol, semI)

    def idx_wait(grow, rcol, semI):
        pltpu.make_async_copy(row.at[pl.ds(0, K)], grow, semI).wait()
        pltpu.make_async_copy(col.at[pl.ds(0, K)], rcol, semI).wait()

    def adjust(grow, rcol, gcol):
        def adj(i, _):
            sl = pl.ds(i * 16, 16)
            grow[sl] = grow[sl] + noff
            gcol[sl] = rcol[sl] + noff
            return 0
        lax.fori_loop(0, K // 16, adj, 0)

    def gat_start(g, grow, gcol, ab, bb, tb, semA, semB, semT):
        e0 = g * K
        pltpu.async_copy(xah.at[grow], ab, semA)
        pltpu.async_copy(xbh.at[gcol], bb, semB)
        pltpu.async_copy(th.at[pl.ds(tbase + e0, K)], tb, semT)

    def gat_wait(grow, gcol, ab, bb, tb, semA, semB, semT):
        pltpu.make_async_copy(xah.at[grow], ab, semA).wait()
        pltpu.make_async_copy(xbh.at[gcol], bb, semB).wait()
        pltpu.make_async_copy(th.at[pl.ds(tbase, K)], tb, semT).wait()

    def comp(ab, bb, tb):
        def body(r, _):
            for f8 in range(HALF // 16):
                sl = pl.ds(f8 * 16, 16)
                tb[r, sl] = jnp.maximum(ab[r, sl] + bb[r, sl] + tb[r, sl],
                                        0.0)
            return 0
        lax.fori_loop(0, K, body, 0)

    def scat(tb, rcol):
        pltpu.sync_copy(tb, acc.at[rcol], add=True)

        @pl.when(c == 0)
        def _():
            pltpu.sync_copy(onesv, degacc.at[rcol], add=True)

    p0 = (rcol0, grow0, gcol0, abuf0, bbuf0, tbuf0, semI0, semA0, semB0, semT0)
    p1 = (rcol1, grow1, gcol1, abuf1, bbuf1, tbuf1, semI1, semA1, semB1, semT1)

    def half(t, mine, other, joff):
        (rc, gr, gc, ab, bb, tb, sI, sA, sB, sT) = mine
        (orc, ogr, ogc, oab, obb, otb, osI, osA, osB, osT) = other
        j = 2 * t + joff
        g_next = (j + 1) * NTILES + s
        g_next2 = (j + 2) * NTILES + s

        @pl.when(j + 1 < NMAIN)
        def _():
            idx_wait(ogr, orc, osI)
            adjust(ogr, orc, ogc)
            gat_start(g_next, ogr, ogc, oab, obb, otb, osA, osB, osT)

        gat_wait(gr, gc, ab, bb, tb, sA, sB, sT)
        comp(ab, bb, tb)
        scat(tb, rc)

        @pl.when(j + 2 < NMAIN)
        def _():
            idx_start(g_next2, gr, rc, sI)

    g0 = s
    pltpu.sync_copy(row.at[pl.ds(g0 * K, K)], grow0)
    pltpu.sync_copy(col.at[pl.ds(g0 * K, K)], rcol0)
    adjust(grow0, rcol0, gcol0)
    gat_start(g0, grow0, gcol0, abuf0, bbuf0, tbuf0, semA0, semB0, semT0)
    idx_start(1 * NTILES + s, grow1, rcol1, semI1)

    def pair(t, _):
        half(t, p0, p1, 0)
        half(t, p1, p0, 1)
        return 0
    lax.fori_loop(0, NPAIR, pair, 0)

    @pl.when(s < NLEFT)
    def _():
        g = NMAIN * NTILES + s
        pltpu.sync_copy(row.at[pl.ds(g * K, K)], grow0)
        pltpu.sync_copy(col.at[pl.ds(g * K, K)], rcol0)
        adjust(grow0, rcol0, gcol0)
        gat_start(g, grow0, gcol0, abuf0, bbuf0, tbuf0, semA0, semB0, semT0)
        gat_wait(grow0, gcol0, abuf0, bbuf0, tbuf0, semA0, semB0, semT0)
        comp(abuf0, bbuf0, tbuf0)
        scat(tbuf0, rcol0)

    plsc.subcore_barrier()

    pltpu.sync_copy(acc.at[pl.ds(s * RPT, RPT)],
                    out_h.at[pl.ds(c * N + s * RPT, RPT)])

    @pl.when(s == NTILES - 1)
    def _():
        pltpu.sync_copy(acc.at[pl.ds(NTILES * RPT, NTAIL)],
                        out_h.at[pl.ds(c * N + NTILES * RPT, NTAIL)])

    @pl.when((s == 0) & (c == 0))
    def _():
        pltpu.sync_copy(degacc, out_deg)


_sc_agg = functools.partial(
    pl.kernel,
    out_type=(jax.ShapeDtypeStruct((NSC * N, HALF), jnp.float32),
              jax.ShapeDtypeStruct((N,), jnp.float32)),
    mesh=plsc.VectorSubcoreMesh(core_axis_name="c", subcore_axis_name="s"),
    scratch_types=(
        [pltpu.VMEM((K,), jnp.int32)] * 6
        + [pltpu.VMEM((K, HALF), jnp.float32)] * 6
        + [pltpu.VMEM((K,), jnp.float32)]
        + [pltpu.VMEM_SHARED((N, HALF), jnp.float32),
           pltpu.VMEM_SHARED((N,), jnp.float32)]
        + [pltpu.SemaphoreType.DMA] * 8
    ),
)(_sc_body)


def _node_mm_body(x_ref, wa_ref, wb_ref, oa_ref, ob_ref):
    x = x_ref[...]
    oa_ref[...] = jnp.dot(x, wa_ref[...],
                          preferred_element_type=jnp.float32)[None]
    ob_ref[...] = jnp.dot(x, wb_ref[...],
                          preferred_element_type=jnp.float32)[None]


def _node_mm(x, wa, wb):
    nb = 1000
    return pl.pallas_call(
        _node_mm_body,
        grid=(NSC, N // nb),
        in_specs=[
            pl.BlockSpec((nb, DN), lambda h, i: (i, 0)),
            pl.BlockSpec((DN, HALF), lambda h, i: (0, h)),
            pl.BlockSpec((DN, HALF), lambda h, i: (0, h)),
        ],
        out_specs=[
            pl.BlockSpec((1, nb, HALF), lambda h, i: (h, i, 0)),
            pl.BlockSpec((1, nb, HALF), lambda h, i: (h, i, 0)),
        ],
        out_shape=[
            jax.ShapeDtypeStruct((NSC, N, HALF), jnp.float32),
            jax.ShapeDtypeStruct((NSC, N, HALF), jnp.float32),
        ],
    )(x, wa, wb)


def _edge_mm_body(e_ref, we_ref, b_ref, o_ref):
    o_ref[...] = (jnp.dot(e_ref[...], we_ref[...],
                          preferred_element_type=jnp.float32)[None]
                  + b_ref[...])


def _edge_mm(e, we, b1h):
    eb = 4000
    return pl.pallas_call(
        _edge_mm_body,
        grid=(NSC, E // eb),
        in_specs=[
            pl.BlockSpec((eb, DE), lambda h, i: (i, 0)),
            pl.BlockSpec((DE, HALF), lambda h, i: (0, h)),
            pl.BlockSpec((1, 1, HALF), lambda h, i: (h, 0, 0)),
        ],
        out_specs=pl.BlockSpec((1, eb, HALF), lambda h, i: (h, i, 0)),
        out_shape=jax.ShapeDtypeStruct((NSC, E, HALF), jnp.float32),
    )(e, we, b1h)


def _wt_body(w2_ref, u1b_ref, b2_ref, v_ref, c_ref):
    u1b = u1b_ref[...]
    v_ref[...] = jnp.dot(w2_ref[...], u1b, preferred_element_type=jnp.float32)
    c_ref[...] = jnp.dot(b2_ref[...], u1b, preferred_element_type=jnp.float32)


def _wt_mm(w2, u1b, b2):
    return pl.pallas_call(
        _wt_body,
        out_shape=[
            jax.ShapeDtypeStruct((HID, HID), jnp.float32),
            jax.ShapeDtypeStruct((1, HID), jnp.float32),
        ],
    )(w2, u1b, b2)


def _post_body(x_ref, h0_ref, h1_ref, d_ref, u1a_ref, v0_ref, v1_ref,
               cv_ref, ub1_ref, u2_ref, ub2_ref, o_ref):
    p = (jnp.dot(x_ref[...], u1a_ref[...], preferred_element_type=jnp.float32)
         + jnp.dot(h0_ref[...], v0_ref[...], preferred_element_type=jnp.float32)
         + jnp.dot(h1_ref[...], v1_ref[...], preferred_element_type=jnp.float32)
         + d_ref[...] * cv_ref[...]
         + ub1_ref[...])
    u = jnp.maximum(p, 0.0)
    o_ref[...] = jnp.dot(u, u2_ref[...],
                         preferred_element_type=jnp.float32) + ub2_ref[...]


def _post_mm(x, h0, h1, deg, u1a, v0, v1, cv, ub1, u2, ub2):
    nb = 1000
    return pl.pallas_call(
        _post_body,
        grid=(N // nb,),
        in_specs=[
            pl.BlockSpec((nb, DN), lambda i: (i, 0)),
            pl.BlockSpec((nb, HALF), lambda i: (i, 0)),
            pl.BlockSpec((nb, HALF), lambda i: (i, 0)),
            pl.BlockSpec((nb, 1), lambda i: (i, 0)),
            pl.BlockSpec((DN, HID), lambda i: (0, 0)),
            pl.BlockSpec((HALF, HID), lambda i: (0, 0)),
            pl.BlockSpec((HALF, HID), lambda i: (0, 0)),
            pl.BlockSpec((1, HID), lambda i: (0, 0)),
            pl.BlockSpec((1, HID), lambda i: (0, 0)),
            pl.BlockSpec((HID, DO), lambda i: (0, 0)),
            pl.BlockSpec((1, DO), lambda i: (0, 0)),
        ],
        out_specs=pl.BlockSpec((nb, DO), lambda i: (i, 0)),
        out_shape=jax.ShapeDtypeStruct((N, DO), jnp.float32),
    )(x, h0, h1, deg, u1a, v0, v1, cv, ub1, u2, ub2)


def kernel(node_features, edge_features, edge_index, W1, b1, W2, b2,
           U1, ub1, U2, ub2):
    row = edge_index[0]
    col = edge_index[1]
    w1a = W1[:DN]
    w1b = W1[DN:2 * DN]
    w1e = W1[2 * DN:]

    xa, xb = _node_mm(node_features, w1a, w1b)
    th = _edge_mm(edge_features, w1e, b1.reshape(NSC, 1, HALF))
    v, cv = _wt_mm(W2, U1[DN:], b2.reshape(1, HID))

    zrows = jnp.zeros((RPT, HALF), jnp.float32)
    zdeg = jnp.zeros((N,), jnp.float32)

    hs, degf = _sc_agg(xa.reshape(NSC * N, HALF), xb.reshape(NSC * N, HALF),
                       th.reshape(NSC * E, HALF), row, col, zrows, zdeg)

    return _post_mm(node_features, hs[:N], hs[N:], degf.reshape(N, 1),
                    U1[:DN], v[:HALF], v[HALF:], cv, ub1.reshape(1, HID),
                    U2, ub2.reshape(1, DO))


# speedup vs baseline: 1.1555x; 1.0061x over previous
import functools
import jax
import jax.numpy as jnp
from jax import lax
from jax.experimental import pallas as pl
from jax.experimental.pallas import tpu as pltpu
from jax.experimental.pallas import tpu_sc as plsc

N = 10000
E = 320000
DN = 128
DE = 16
HID = 256
DO = 128
HALF = 128
NSC = 2
NTILES = 16
K = 64
GCHUNKS = E // K
NMAIN = 312
NPAIR = NMAIN // 2
NLEFT = GCHUNKS - NMAIN * NTILES
RPT = 624
NTAIL = N - NTILES * RPT


def _sc_body(xah, xbh, th, row, col, zrows, zdeg, out_h, out_deg,
             rcol0, grow0, gcol0, rcol1, grow1, gcol1,
             abuf0, bbuf0, tbuf0, abuf1, bbuf1, tbuf1, onesv,
             acc, degacc,
             semI0, semI1, semA0, semB0, semT0, semA1, semB1, semT1):
    c = lax.axis_index("c")
    s = lax.axis_index("s")

    pltpu.sync_copy(zrows, acc.at[pl.ds(s * RPT, RPT)])

    @pl.when(s == NTILES - 1)
    def _():
        pltpu.sync_copy(zrows.at[pl.ds(0, NTAIL)],
                        acc.at[pl.ds(NTILES * RPT, NTAIL)])

    @pl.when(s == 0)
    def _():
        pltpu.sync_copy(zdeg, degacc)

    def initones(i, _):
        onesv[pl.ds(i * 16, 16)] = jnp.full((16,), 1.0, jnp.float32)
        return 0
    lax.fori_loop(0, K // 16, initones, 0)

    plsc.subcore_barrier()

    tbase = c * E
    noff = c * N

    def idx_start(g, grow, rcol, semI):
        e0 = g * K
        pltpu.async_copy(row.at[pl.ds(e0, K)], grow, semI)
        pltpu.async_copy(col.at[pl.ds(e0, K)], rcol, semI)

    def idx_wait(grow, rcol, semI):
        pltpu.make_async_copy(row.at[pl.ds(0, K)], grow, semI).wait()
        pltpu.make_async_copy(col.at[pl.ds(0, K)], rcol, semI).wait()

    def adjust(grow, rcol, gcol):
        def adj(i, _):
            sl = pl.ds(i * 16, 16)
            grow[sl] = grow[sl] + noff
            gcol[sl] = rcol[sl] + noff
            return 0
        lax.fori_loop(0, K // 16, adj, 0)

    def gat_start(g, grow, gcol, ab, bb, tb, semA, semB, semT):
        e0 = g * K
        pltpu.async_copy(xah.at[grow], ab, semA)
        pltpu.async_copy(xbh.at[gcol], bb, semB)
        pltpu.async_copy(th.at[pl.ds(tbase + e0, K)], tb, semT)

    def gat_wait(grow, gcol, ab, bb, tb, semA, semB, semT):
        pltpu.make_async_copy(xah.at[grow], ab, semA).wait()
        pltpu.make_async_copy(xbh.at[gcol], bb, semB).wait()
        pltpu.make_async_copy(th.at[pl.ds(tbase, K)], tb, semT).wait()

    def comp(ab, bb, tb):
        def body(r, _):
            for f8 in range(HALF // 16):
                sl = pl.ds(f8 * 16, 16)
                tb[r, sl] = jnp.maximum(ab[r, sl] + bb[r, sl] + tb[r, sl],
                                        0.0)
            return 0
        lax.fori_loop(0, K, body, 0)

    def scat(tb, rcol):
        pltpu.sync_copy(tb, acc.at[rcol], add=True)

        @pl.when(c == 0)
        def _():
            pltpu.sync_copy(onesv, degacc.at[rcol], add=True)

    p0 = (rcol0, grow0, gcol0, abuf0, bbuf0, tbuf0, semI0, semA0, semB0, semT0)
    p1 = (rcol1, grow1, gcol1, abuf1, bbuf1, tbuf1, semI1, semA1, semB1, semT1)

    def half(t, mine, other, joff):
        (rc, gr, gc, ab, bb, tb, sI, sA, sB, sT) = mine
        (orc, ogr, ogc, oab, obb, otb, osI, osA, osB, osT) = other
        j = 2 * t + joff
        g_next = (j + 1) * NTILES + s
        g_next2 = (j + 2) * NTILES + s

        @pl.when(j + 1 < NMAIN)
        def _():
            idx_wait(ogr, orc, osI)
            adjust(ogr, orc, ogc)
            gat_start(g_next, ogr, ogc, oab, obb, otb, osA, osB, osT)

        gat_wait(gr, gc, ab, bb, tb, sA, sB, sT)
        comp(ab, bb, tb)
        scat(tb, rc)

        @pl.when(j + 2 < NMAIN)
        def _():
            idx_start(g_next2, gr, rc, sI)

    g0 = s
    pltpu.sync_copy(row.at[pl.ds(g0 * K, K)], grow0)
    pltpu.sync_copy(col.at[pl.ds(g0 * K, K)], rcol0)
    adjust(grow0, rcol0, gcol0)
    gat_start(g0, grow0, gcol0, abuf0, bbuf0, tbuf0, semA0, semB0, semT0)
    idx_start(1 * NTILES + s, grow1, rcol1, semI1)

    def pair(t, _):
        half(t, p0, p1, 0)
        half(t, p1, p0, 1)
        return 0
    lax.fori_loop(0, NPAIR, pair, 0)

    @pl.when(s < NLEFT)
    def _():
        g = NMAIN * NTILES + s
        pltpu.sync_copy(row.at[pl.ds(g * K, K)], grow0)
        pltpu.sync_copy(col.at[pl.ds(g * K, K)], rcol0)
        adjust(grow0, rcol0, gcol0)
        gat_start(g, grow0, gcol0, abuf0, bbuf0, tbuf0, semA0, semB0, semT0)
        gat_wait(grow0, gcol0, abuf0, bbuf0, tbuf0, semA0, semB0, semT0)
        comp(abuf0, bbuf0, tbuf0)
        scat(tbuf0, rcol0)

    plsc.subcore_barrier()

    pltpu.sync_copy(acc.at[pl.ds(s * RPT, RPT)],
                    out_h.at[pl.ds(c * N + s * RPT, RPT)])

    @pl.when(s == NTILES - 1)
    def _():
        pltpu.sync_copy(acc.at[pl.ds(NTILES * RPT, NTAIL)],
                        out_h.at[pl.ds(c * N + NTILES * RPT, NTAIL)])

    @pl.when((s == 0) & (c == 0))
    def _():
        pltpu.sync_copy(degacc, out_deg)


_sc_agg = functools.partial(
    pl.kernel,
    out_type=(jax.ShapeDtypeStruct((NSC * N, HALF), jnp.float32),
              jax.ShapeDtypeStruct((N,), jnp.float32)),
    mesh=plsc.VectorSubcoreMesh(core_axis_name="c", subcore_axis_name="s"),
    scratch_types=(
        [pltpu.VMEM((K,), jnp.int32)] * 6
        + [pltpu.VMEM((K, HALF), jnp.float32)] * 6
        + [pltpu.VMEM((K,), jnp.float32)]
        + [pltpu.VMEM_SHARED((N, HALF), jnp.float32),
           pltpu.VMEM_SHARED((N,), jnp.float32)]
        + [pltpu.SemaphoreType.DMA] * 8
    ),
)(_sc_body)


def _node_mm_body(x_ref, wa_ref, wb_ref, oa_ref, ob_ref):
    x = x_ref[...]
    oa_ref[...] = jnp.dot(x, wa_ref[...],
                          preferred_element_type=jnp.float32)[None]
    ob_ref[...] = jnp.dot(x, wb_ref[...],
                          preferred_element_type=jnp.float32)[None]


def _node_mm(x, wa, wb):
    nb = 1000
    return pl.pallas_call(
        _node_mm_body,
        grid=(NSC, N // nb),
        in_specs=[
            pl.BlockSpec((nb, DN), lambda h, i: (i, 0)),
            pl.BlockSpec((DN, HALF), lambda h, i: (0, h)),
            pl.BlockSpec((DN, HALF), lambda h, i: (0, h)),
        ],
        out_specs=[
            pl.BlockSpec((1, nb, HALF), lambda h, i: (h, i, 0)),
            pl.BlockSpec((1, nb, HALF), lambda h, i: (h, i, 0)),
        ],
        out_shape=[
            jax.ShapeDtypeStruct((NSC, N, HALF), jnp.float32),
            jax.ShapeDtypeStruct((NSC, N, HALF), jnp.float32),
        ],
    )(x, wa, wb)


def _edge_mm_body(e_ref, we_ref, b_ref, o_ref):
    o_ref[...] = (jnp.dot(e_ref[...], we_ref[...],
                          preferred_element_type=jnp.float32)[None]
                  + b_ref[...])


def _edge_mm(e, we, b1h):
    eb = 4000
    return pl.pallas_call(
        _edge_mm_body,
        grid=(NSC, E // eb),
        in_specs=[
            pl.BlockSpec((eb, DE), lambda h, i: (i, 0)),
            pl.BlockSpec((DE, HALF), lambda h, i: (0, h)),
            pl.BlockSpec((1, 1, HALF), lambda h, i: (h, 0, 0)),
        ],
        out_specs=pl.BlockSpec((1, eb, HALF), lambda h, i: (h, i, 0)),
        out_shape=jax.ShapeDtypeStruct((NSC, E, HALF), jnp.float32),
    )(e, we, b1h)


def _wt_body(w2_ref, u1b_ref, b2_ref, v_ref, c_ref):
    u1b = u1b_ref[...]
    v_ref[...] = jnp.dot(w2_ref[...], u1b, preferred_element_type=jnp.float32)
    c_ref[...] = jnp.dot(b2_ref[...], u1b, preferred_element_type=jnp.float32)


def _wt_mm(w2, u1b, b2):
    return pl.pallas_call(
        _wt_body,
        out_shape=[
            jax.ShapeDtypeStruct((HID, HID), jnp.float32),
            jax.ShapeDtypeStruct((1, HID), jnp.float32),
        ],
    )(w2, u1b, b2)


def _post_body(x_ref, h0_ref, h1_ref, d_ref, u1a_ref, v0_ref, v1_ref,
               cv_ref, ub1_ref, u2_ref, ub2_ref, o_ref):
    p = (jnp.dot(x_ref[...], u1a_ref[...], preferred_element_type=jnp.float32)
         + jnp.dot(h0_ref[...], v0_ref[...], preferred_element_type=jnp.float32)
         + jnp.dot(h1_ref[...], v1_ref[...], preferred_element_type=jnp.float32)
         + d_ref[...] * cv_ref[...]
         + ub1_ref[...])
    u = jnp.maximum(p, 0.0)
    o_ref[...] = jnp.dot(u, u2_ref[...],
                         preferred_element_type=jnp.float32) + ub2_ref[...]


def _post_mm(x, hs, deg, u1a, v0, v1, cv, ub1, u2, ub2):
    nb = 1000
    return pl.pallas_call(
        _post_body,
        grid=(N // nb,),
        in_specs=[
            pl.BlockSpec((nb, DN), lambda i: (i, 0)),
            pl.BlockSpec((nb, HALF), lambda i: (i, 0)),
            pl.BlockSpec((nb, HALF), lambda i: (N // nb + i, 0)),
            pl.BlockSpec((nb, 1), lambda i: (i, 0)),
            pl.BlockSpec((DN, HID), lambda i: (0, 0)),
            pl.BlockSpec((HALF, HID), lambda i: (0, 0)),
            pl.BlockSpec((HALF, HID), lambda i: (0, 0)),
            pl.BlockSpec((1, HID), lambda i: (0, 0)),
            pl.BlockSpec((1, HID), lambda i: (0, 0)),
            pl.BlockSpec((HID, DO), lambda i: (0, 0)),
            pl.BlockSpec((1, DO), lambda i: (0, 0)),
        ],
        out_specs=pl.BlockSpec((nb, DO), lambda i: (i, 0)),
        out_shape=jax.ShapeDtypeStruct((N, DO), jnp.float32),
    )(x, hs, hs, deg, u1a, v0, v1, cv, ub1, u2, ub2)


def kernel(node_features, edge_features, edge_index, W1, b1, W2, b2,
           U1, ub1, U2, ub2):
    row = edge_index[0]
    col = edge_index[1]
    w1a = W1[:DN]
    w1b = W1[DN:2 * DN]
    w1e = W1[2 * DN:]

    xa, xb = _node_mm(node_features, w1a, w1b)
    th = _edge_mm(edge_features, w1e, b1.reshape(NSC, 1, HALF))
    v, cv = _wt_mm(W2, U1[DN:], b2.reshape(1, HID))

    zrows = jnp.zeros((RPT, HALF), jnp.float32)
    zdeg = jnp.zeros((N,), jnp.float32)

    hs, degf = _sc_agg(xa.reshape(NSC * N, HALF), xb.reshape(NSC * N, HALF),
                       th.reshape(NSC * E, HALF), row, col, zrows, zdeg)

    return _post_mm(node_features, hs, degf.reshape(N, 1),
                    U1[:DN], v[:HALF], v[HALF:], cv, ub1.reshape(1, HID),
                    U2, ub2.reshape(1, DO))
